# baseline (speedup 1.0000x reference)
import functools

import jax
import jax.numpy as jnp
from jax import lax
from jax.experimental import pallas as pl
from jax.experimental.pallas import tpu as pltpu
from jax.experimental.pallas import tpu_sc as plsc

N_NODES = 10000
D = 128
E = 320000

NC = 2
NS = 16
L = 16
NW = NC * NS
EPW = E // NW
CHUNK = 80
NCHUNK = EPW // CHUNK
NGRP = CHUNK // L
DV = D // L

ROW_BLOCK = 2000


def _dense_body(xc_ref, xp_ref, wc_ref, bc_ref, wp_ref, bp_ref,
                wl_ref, bl_ref, wr_ref, br_ref, oc_ref, op_ref):
  inv_s = 1.0 / jnp.sqrt(jnp.float32(D))
  hc = jnp.maximum(
      jnp.dot(xc_ref[...], wc_ref[...], preferred_element_type=jnp.float32)
      + bc_ref[...], 0.0)
  oc_ref[...] = (
      jnp.dot(hc, wl_ref[...], preferred_element_type=jnp.float32)
      + bl_ref[...]) * inv_s
  hp = jnp.maximum(
      jnp.dot(xp_ref[...], wp_ref[...], preferred_element_type=jnp.float32)
      + bp_ref[...], 0.0)
  op_ref[...] = (
      jnp.dot(hp, wr_ref[...], preferred_element_type=jnp.float32)
      + br_ref[...])


def _dense_project(xc, xp, wc, bc, wp, bp, wl, bl, wr, br):
  nb = N_NODES // ROW_BLOCK
  row_spec = pl.BlockSpec((ROW_BLOCK, D), lambda i: (i, 0))
  w_spec = pl.BlockSpec((D, D), lambda i: (0, 0))
  b_spec = pl.BlockSpec((1, D), lambda i: (0, 0))
  return pl.pallas_call(
      _dense_body,
      grid=(nb,),
      in_specs=[row_spec, row_spec, w_spec, b_spec, w_spec, b_spec,
                w_spec, b_spec, w_spec, b_spec],
      out_specs=[row_spec, row_spec],
      out_shape=[jax.ShapeDtypeStruct((N_NODES, D), jnp.float32),
                 jax.ShapeDtypeStruct((N_NODES, D), jnp.float32)],
  )(xc, xp, wc, bc, wp, bp, wl, bl, wr, br)


NBUF = 5
NSPLIT = 1
CSUB = CHUNK // NSPLIT
NMAIN = NCHUNK // NBUF
NTAIL = NCHUNK - NBUF * NMAIN


def _edge_dot_body(cmp_hbm, prt_hbm, eli_hbm, out_hbm,
                   row_all, col_all,
                   cmp_b0, cmp_b1, cmp_b2, cmp_b3, cmp_b4,
                   prt_b0, prt_b1, prt_b2, prt_b3, prt_b4,
                   out_s0, out_s1, out_s2, out_s3, out_s4,
                   sem0, sem1, sem2, sem3, sem4,
                   osem0, osem1, osem2, osem3, osem4):
  wid = lax.axis_index("s") * NC + lax.axis_index("c")
  base = wid * EPW
  cmp_bufs = [cmp_b0, cmp_b1, cmp_b2, cmp_b3, cmp_b4]
  prt_bufs = [prt_b0, prt_b1, prt_b2, prt_b3, prt_b4]
  out_stg = [out_s0, out_s1, out_s2, out_s3, out_s4]
  sems = [sem0, sem1, sem2, sem3, sem4]
  osems = [osem0, osem1, osem2, osem3, osem4]
  lanes = lax.iota(jnp.int32, L)
  skew = [(lanes + t) & (L - 1) for t in range(L)]

  pltpu.sync_copy(eli_hbm.at[pl.ds(base, EPW)], row_all)
  pltpu.sync_copy(eli_hbm.at[pl.ds(E + base, EPW)], col_all)

  def issue(c, b):
    off = c * CHUNK
    for h in range(NSPLIT):
      pltpu.async_copy(
          cmp_hbm.at[row_all.at[pl.ds(off + h * CSUB, CSUB)]],
          cmp_bufs[b].at[pl.ds(h * CSUB, CSUB), :], sems[b])
      pltpu.async_copy(
          prt_hbm.at[col_all.at[pl.ds(off + h * CSUB, CSUB)]],
          prt_bufs[b].at[pl.ds(h * CSUB, CSUB), :], sems[b])

  def wait(b):
    for h in range(NSPLIT):
      pltpu.make_async_copy(
          cmp_hbm.at[pl.ds(0, CSUB), :],
          cmp_bufs[b].at[pl.ds(h * CSUB, CSUB), :], sems[b]).wait()
      pltpu.make_async_copy(
          prt_hbm.at[pl.ds(0, CSUB), :],
          prt_bufs[b].at[pl.ds(h * CSUB, CSUB), :], sems[b]).wait()

  def compute(c, b):
    cmp_v = cmp_bufs[b]
    prt_v = prt_bufs[b]
    o_v = out_stg[b]

    @pl.when(c >= NBUF)
    def _():
      pltpu.make_async_copy(
          o_v.at[pl.ds(0, CHUNK)],
          out_hbm.at[pl.ds(0, CHUNK)], osems[b]).wait()

    zero = jnp.zeros((L,), jnp.float32)
    for g in range(NGRP):
      rows = lanes + (g * L)

      def dd_body(dd, accs, rows=rows):
        base = dd * L
        accs = list(accs)
        for t in range(L):
          cols = skew[t] + base
          p = (plsc.load_gather(cmp_v, [rows, cols])
               * plsc.load_gather(prt_v, [rows, cols]))
          accs[t & 3] = accs[t & 3] + p
        return tuple(accs)

      a0, a1, a2, a3 = lax.fori_loop(
          0, DV, dd_body, (zero, zero, zero, zero))
      o_v[pl.ds(g * L, L)] = (a0 + a1) + (a2 + a3)
    pltpu.async_copy(
        o_v.at[pl.ds(0, CHUNK)],
        out_hbm.at[pl.ds(base + c * CHUNK, CHUNK)], osems[b])

  for b in range(NBUF):
    issue(b, b)

  def outer(o, carry):
    for b in range(NBUF):
      c = o * NBUF + b
      wait(b)
      compute(c, b)
      nxt = c + NBUF

      @pl.when(nxt < NCHUNK)
      def _():
        issue(nxt, b)

    return carry

  lax.fori_loop(0, NMAIN, outer, 0)
  for b in range(NTAIL):
    c = NBUF * NMAIN + b
    wait(b)
    compute(c, b)
  for b in range(NBUF):
    pltpu.make_async_copy(
        out_stg[b].at[pl.ds(0, CHUNK)],
        out_hbm.at[pl.ds(0, CHUNK)], osems[b]).wait()


_edge_dot = functools.partial(
    pl.kernel,
    out_type=jax.ShapeDtypeStruct((E,), jnp.float32),
    mesh=plsc.VectorSubcoreMesh(core_axis_name="c", subcore_axis_name="s"),
    compiler_params=pltpu.CompilerParams(needs_layout_passes=False),
    scratch_types=(
        [pltpu.VMEM((EPW,), jnp.int32),
         pltpu.VMEM((EPW,), jnp.int32)]
        + [pltpu.VMEM((CHUNK, D), jnp.float32) for _ in range(2 * NBUF)]
        + [pltpu.VMEM((CHUNK + L,), jnp.float32) for _ in range(NBUF)]
        + [pltpu.SemaphoreType.DMA for _ in range(2 * NBUF)]
    ),
)(_edge_dot_body)


@jax.jit
def kernel(x_compound, x_protein, edge_index_cp, edge_label_index,
           W_compound, b_compound, W_protein, b_protein,
           W_left, b_left, W_right, b_right):
  cmp_lin, prt_lin = _dense_project(
      x_compound, x_protein,
      W_compound, b_compound.reshape(1, D),
      W_protein, b_protein.reshape(1, D),
      W_left, b_left.reshape(1, D),
      W_right, b_right.reshape(1, D))
  return _edge_dot(cmp_lin, prt_lin, edge_label_index.reshape(2 * E))


# speedup vs baseline: 1.0204x; 1.0204x over previous
import functools

import jax
import jax.numpy as jnp
from jax import lax
from jax.experimental import pallas as pl
from jax.experimental.pallas import tpu as pltpu
from jax.experimental.pallas import tpu_sc as plsc

N_NODES = 10000
D = 128
E = 320000

NC = 2
NS = 16
L = 16
NW = NC * NS
EPW = E // NW
CHUNK = 80
NCHUNK = EPW // CHUNK
NGRP = CHUNK // L
DV = D // L

ROW_BLOCK = 2000


def _dense_body(xc_ref, xp_ref, wc_ref, bc_ref, wp_ref, bp_ref,
                wl_ref, bl_ref, wr_ref, br_ref, oc_ref, op_ref):
  inv_s = 1.0 / jnp.sqrt(jnp.float32(D))
  hc = jnp.maximum(
      jnp.dot(xc_ref[...], wc_ref[...], preferred_element_type=jnp.float32)
      + bc_ref[...], 0.0)
  oc_ref[...] = (
      jnp.dot(hc, wl_ref[...], preferred_element_type=jnp.float32)
      + bl_ref[...]) * inv_s
  hp = jnp.maximum(
      jnp.dot(xp_ref[...], wp_ref[...], preferred_element_type=jnp.float32)
      + bp_ref[...], 0.0)
  op_ref[...] = (
      jnp.dot(hp, wr_ref[...], preferred_element_type=jnp.float32)
      + br_ref[...])


def _dense_project(xc, xp, wc, bc, wp, bp, wl, bl, wr, br):
  nb = N_NODES // ROW_BLOCK
  row_spec = pl.BlockSpec((ROW_BLOCK, D), lambda i: (i, 0))
  w_spec = pl.BlockSpec((D, D), lambda i: (0, 0))
  b_spec = pl.BlockSpec((1, D), lambda i: (0, 0))
  return pl.pallas_call(
      _dense_body,
      grid=(nb,),
      in_specs=[row_spec, row_spec, w_spec, b_spec, w_spec, b_spec,
                w_spec, b_spec, w_spec, b_spec],
      out_specs=[row_spec, row_spec],
      out_shape=[jax.ShapeDtypeStruct((N_NODES, D), jnp.float32),
                 jax.ShapeDtypeStruct((N_NODES, D), jnp.float32)],
  )(xc, xp, wc, bc, wp, bp, wl, bl, wr, br)


NBUF = 4
NSPLIT = 1
CSUB = CHUNK // NSPLIT
NMAIN = NCHUNK // NBUF
NTAIL = NCHUNK - NBUF * NMAIN


def _edge_dot_body(cmp_hbm, prt_hbm, eli_hbm, out_hbm,
                   row_all, col_all,
                   cmp_b0, cmp_b1, cmp_b2, cmp_b3,
                   prt_b0, prt_b1, prt_b2, prt_b3,
                   out_s0, out_s1, out_s2, out_s3,
                   sem0, sem1, sem2, sem3,
                   osem0, osem1, osem2, osem3):
  wid = lax.axis_index("s") * NC + lax.axis_index("c")
  base = wid * EPW
  cmp_bufs = [cmp_b0, cmp_b1, cmp_b2, cmp_b3]
  prt_bufs = [prt_b0, prt_b1, prt_b2, prt_b3]
  out_stg = [out_s0, out_s1, out_s2, out_s3]
  sems = [sem0, sem1, sem2, sem3]
  osems = [osem0, osem1, osem2, osem3]
  lanes = lax.iota(jnp.int32, L)
  skew = [(lanes + t) & (L - 1) for t in range(L)]

  pltpu.sync_copy(eli_hbm.at[pl.ds(base, EPW)], row_all)
  pltpu.sync_copy(eli_hbm.at[pl.ds(E + base, EPW)], col_all)

  def issue(c, b):
    off = c * CHUNK
    for h in range(NSPLIT):
      pltpu.async_copy(
          cmp_hbm.at[row_all.at[pl.ds(off + h * CSUB, CSUB)]],
          cmp_bufs[b].at[pl.ds(h * CSUB, CSUB), :], sems[b])
      pltpu.async_copy(
          prt_hbm.at[col_all.at[pl.ds(off + h * CSUB, CSUB)]],
          prt_bufs[b].at[pl.ds(h * CSUB, CSUB), :], sems[b])

  def wait(b):
    for h in range(NSPLIT):
      pltpu.make_async_copy(
          cmp_hbm.at[pl.ds(0, CSUB), :],
          cmp_bufs[b].at[pl.ds(h * CSUB, CSUB), :], sems[b]).wait()
      pltpu.make_async_copy(
          prt_hbm.at[pl.ds(0, CSUB), :],
          prt_bufs[b].at[pl.ds(h * CSUB, CSUB), :], sems[b]).wait()

  def compute(c, b):
    cmp_v = cmp_bufs[b]
    prt_v = prt_bufs[b]
    o_v = out_stg[b]

    @pl.when(c >= NBUF)
    def _():
      pltpu.make_async_copy(
          o_v.at[pl.ds(0, CHUNK)],
          out_hbm.at[pl.ds(0, CHUNK)], osems[b]).wait()

    zero = jnp.zeros((L,), jnp.float32)
    for g in range(NGRP):
      rows = lanes + (g * L)

      def dd_body(dd, accs, rows=rows):
        base = dd * L
        accs = list(accs)
        for t in range(L):
          cols = skew[t] + base
          p = (plsc.load_gather(cmp_v, [rows, cols])
               * plsc.load_gather(prt_v, [rows, cols]))
          accs[t & 3] = accs[t & 3] + p
        return tuple(accs)

      a0, a1, a2, a3 = lax.fori_loop(
          0, DV, dd_body, (zero, zero, zero, zero))
      o_v[pl.ds(g * L, L)] = (a0 + a1) + (a2 + a3)
    pltpu.async_copy(
        o_v.at[pl.ds(0, CHUNK)],
        out_hbm.at[pl.ds(base + c * CHUNK, CHUNK)], osems[b])

  for b in range(NBUF):
    issue(b, b)

  def outer(o, carry):
    for b in range(NBUF):
      c = o * NBUF + b
      wait(b)
      compute(c, b)
      nxt = c + NBUF

      @pl.when(nxt < NCHUNK)
      def _():
        issue(nxt, b)

    return carry

  lax.fori_loop(0, NMAIN, outer, 0)
  for b in range(NTAIL):
    c = NBUF * NMAIN + b
    wait(b)
    compute(c, b)
  for b in range(NBUF):
    pltpu.make_async_copy(
        out_stg[b].at[pl.ds(0, CHUNK)],
        out_hbm.at[pl.ds(0, CHUNK)], osems[b]).wait()


_edge_dot = functools.partial(
    pl.kernel,
    out_type=jax.ShapeDtypeStruct((E,), jnp.float32),
    mesh=plsc.VectorSubcoreMesh(core_axis_name="c", subcore_axis_name="s"),
    compiler_params=pltpu.CompilerParams(needs_layout_passes=False),
    scratch_types=(
        [pltpu.VMEM((EPW,), jnp.int32),
         pltpu.VMEM((EPW,), jnp.int32)]
        + [pltpu.VMEM((CHUNK, D), jnp.float32) for _ in range(2 * NBUF)]
        + [pltpu.VMEM((CHUNK + L,), jnp.float32) for _ in range(NBUF)]
        + [pltpu.SemaphoreType.DMA for _ in range(2 * NBUF)]
    ),
)(_edge_dot_body)


@jax.jit
def kernel(x_compound, x_protein, edge_index_cp, edge_label_index,
           W_compound, b_compound, W_protein, b_protein,
           W_left, b_left, W_right, b_right):
  cmp_lin, prt_lin = _dense_project(
      x_compound, x_protein,
      W_compound, b_compound.reshape(1, D),
      W_protein, b_protein.reshape(1, D),
      W_left, b_left.reshape(1, D),
      W_right, b_right.reshape(1, D))
  return _edge_dot(cmp_lin, prt_lin, edge_label_index.reshape(2 * E))
